# speedup vs baseline: 1936.6721x; 1936.6721x over previous
import jax
import jax.numpy as jnp
from jax.experimental import pallas as pl

NPAD = 1024
H = 128


def _leaky(v):
    return jnp.where(v >= 0, v, 0.01 * v)


def _gnn_last_step_kernel(x_ref, adj_ref, w1_ref, b1_ref, w2_ref, b2_ref,
                          wrel_ref, brel_ref, wroot_ref, w3_ref, b3_ref,
                          w4_ref, b4_ref, ones_ref, out_ref):
    x = x_ref[...]
    h1 = _leaky(jnp.dot(x, w1_ref[...], preferred_element_type=jnp.float32)
                + b1_ref[...])
    h2 = _leaky(jnp.dot(h1, w2_ref[...], preferred_element_type=jnp.float32)
                + b2_ref[...])

    a = (adj_ref[...] != 0).astype(jnp.float32)
    agg = jax.lax.dot_general(a, h2, (((0,), (0,)), ((), ())),
                              preferred_element_type=jnp.float32)
    cnt = jax.lax.dot_general(a, ones_ref[...], (((0,), (0,)), ((), ())),
                              preferred_element_type=jnp.float32)
    mean = agg / jnp.maximum(cnt, 1.0)

    conv = _leaky(jnp.dot(mean, wrel_ref[...], preferred_element_type=jnp.float32)
                  + brel_ref[...]
                  + jnp.dot(h2, wroot_ref[...], preferred_element_type=jnp.float32))
    h3 = _leaky(jnp.dot(conv, w3_ref[...], preferred_element_type=jnp.float32)
                + b3_ref[...])
    y = jnp.dot(h3, w4_ref[...], preferred_element_type=jnp.float32) + b4_ref[...]
    out_ref[...] = y


def kernel(x, edge_indexs, edgenum, W_fc1, b_fc1, W_fc2, b_fc2, W_rel, b_rel,
           W_root, W_fc3, b_fc3, W_fc4, b_fc4):
    batch, seq_len, n, _ = x.shape

    x_last = jnp.pad(x[0, -1], ((0, NPAD - n), (0, 0)))
    adj_last = jnp.pad(edge_indexs[0, -1], ((0, NPAD - n), (0, NPAD - n)))
    ones_col = jnp.ones((NPAD, H), dtype=jnp.float32)

    y = pl.pallas_call(
        _gnn_last_step_kernel,
        out_shape=jax.ShapeDtypeStruct((NPAD, H), jnp.float32),
    )(x_last, adj_last,
      W_fc1.T, b_fc1[None, :], W_fc2.T, b_fc2[None, :],
      W_rel.T, b_rel[None, :], W_root.T,
      W_fc3.T, b_fc3[None, :],
      jnp.pad(W_fc4.T, ((0, 0), (0, H - 1))),
      jnp.pad(b_fc4[None, :], ((0, 0), (0, H - 1))),
      ones_col)

    return y[:n, :1].reshape(batch, n, 1)


# speedup vs baseline: 4442.2244x; 2.2937x over previous
import jax
import jax.numpy as jnp
from jax.experimental import pallas as pl


def _leaky(v):
    return jnp.where(v >= 0, v, 0.01 * v)


def _tr(a, b):
    return jax.lax.dot_general(a, b, (((1,), (1,)), ((), ())),
                               preferred_element_type=jnp.float32)


def _gnn_last_step_kernel(x_ref, adj_ref, w1_ref, b1_ref, w2_ref, b2_ref,
                          wrel_ref, brel_ref, wroot_ref, w3_ref, b3_ref,
                          w4_ref, b4_ref, ones_ref, out_ref):
    x = x_ref[0, 0]
    h1 = _leaky(_tr(x, w1_ref[...]) + b1_ref[...])
    h2 = _leaky(_tr(h1, w2_ref[...]) + b2_ref[...])

    a = (adj_ref[0, 0] != 0).astype(jnp.float32)
    agg = jax.lax.dot_general(a, h2, (((0,), (0,)), ((), ())),
                              preferred_element_type=jnp.float32)
    cnt = jax.lax.dot_general(a, ones_ref[...], (((0,), (0,)), ((), ())),
                              preferred_element_type=jnp.float32)
    mean = agg / jnp.maximum(cnt, 1.0)

    conv = _leaky(_tr(mean, wrel_ref[...]) + brel_ref[...]
                  + _tr(h2, wroot_ref[...]))
    h3 = _leaky(_tr(conv, w3_ref[...]) + b3_ref[...])
    y = jnp.sum(h3 * w4_ref[...], axis=1, keepdims=True) + b4_ref[...]
    out_ref[...] = y


def kernel(x, edge_indexs, edgenum, W_fc1, b_fc1, W_fc2, b_fc2, W_rel, b_rel,
           W_root, W_fc3, b_fc3, W_fc4, b_fc4):
    batch, seq_len, n, f_in = x.shape
    h = W_fc1.shape[0]
    last = seq_len - 1

    def full(shape):
        return pl.BlockSpec(shape, lambda i: tuple(0 for _ in shape))

    y = pl.pallas_call(
        _gnn_last_step_kernel,
        out_shape=jax.ShapeDtypeStruct((n, 1), jnp.float32),
        grid=(1,),
        in_specs=[
            pl.BlockSpec((1, 1, n, f_in), lambda i: (0, last, 0, 0)),
            pl.BlockSpec((1, 1, n, n), lambda i: (0, last, 0, 0)),
            full((h, f_in)), full((1, h)),
            full((h, h)), full((1, h)),
            full((h, h)), full((1, h)),
            full((h, h)),
            full((h, h)), full((1, h)),
            full((1, h)), full((1, 1)),
            full((n, h)),
        ],
        out_specs=pl.BlockSpec((n, 1), lambda i: (0, 0)),
    )(x, edge_indexs,
      W_fc1, b_fc1[None, :], W_fc2, b_fc2[None, :],
      W_rel, b_rel[None, :], W_root,
      W_fc3, b_fc3[None, :],
      W_fc4, b_fc4[None, :],
      jnp.ones((n, h), dtype=jnp.float32))

    return y.reshape(batch, n, 1)
